# baseline (speedup 1.0000x reference)
import functools

import jax
import jax.numpy as jnp
from jax import lax
from jax.experimental import pallas as pl
from jax.experimental.pallas import tpu as pltpu
from jax.experimental.pallas import tpu_sc as plsc

_MESH = plsc.VectorSubcoreMesh(core_axis_name="c", subcore_axis_name="s")


@functools.partial(
    pl.kernel,
    out_type=jax.ShapeDtypeStruct((8,), jnp.float32),
    mesh=_MESH,
    scratch_types=[pltpu.VMEM((8,), jnp.float32)],
)
def _take_head_sc(x_hbm, out_hbm, buf):
    c = lax.axis_index("c")
    s = lax.axis_index("s")

    @pl.when(jnp.logical_and(c == 0, s == 0))
    def _():
        pltpu.sync_copy(x_hbm.at[pl.ds(0, 8)], buf)
        pltpu.sync_copy(buf, out_hbm)


def kernel(x):
    return _take_head_sc(x)[0]


# speedup vs baseline: 1.0455x; 1.0455x over previous
import functools

import jax
import jax.numpy as jnp
from jax import lax
from jax.experimental import pallas as pl
from jax.experimental.pallas import tpu as pltpu
from jax.experimental.pallas import tpu_sc as plsc

_MESH = plsc.VectorSubcoreMesh(
    core_axis_name="c", subcore_axis_name="s", num_cores=1, num_subcores=1
)


@functools.partial(
    pl.kernel,
    out_type=jax.ShapeDtypeStruct((1,), jnp.float32),
    mesh=_MESH,
    scratch_types=[pltpu.VMEM((1,), jnp.float32)],
)
def _take_first_sc(x_hbm, out_hbm, buf):
    pltpu.sync_copy(x_hbm.at[pl.ds(0, 1)], buf)
    pltpu.sync_copy(buf, out_hbm)


def kernel(x):
    return _take_first_sc(x).reshape(())


# speedup vs baseline: 14.0578x; 13.4465x over previous
import jax
import jax.numpy as jnp
from jax.experimental import pallas as pl
from jax.experimental.pallas import tpu as pltpu


def _take_first(x_ref, o_ref):
    o_ref[0] = x_ref[0]


def kernel(x):
    out = pl.pallas_call(
        _take_first,
        out_shape=jax.ShapeDtypeStruct((1,), jnp.float32),
        grid=(1,),
        in_specs=[pl.BlockSpec((128,), lambda i: (0,), memory_space=pltpu.SMEM)],
        out_specs=pl.BlockSpec((1,), lambda i: (0,), memory_space=pltpu.SMEM),
    )(x)
    return out.reshape(())
